# Initial kernel scaffold; baseline (speedup 1.0000x reference)
import jax
import jax.numpy as jnp
from jax.experimental import pallas as pl


def kernel(x, edge_index, edge_attr, batch, atom_tables, We, be, Wp, bp, W1, b1, W2, b2, W3, b3):
    raise NotImplementedError("write your pallas kernel here")


# speedup vs baseline: 1.0082x; 1.0082x over previous
import functools

import jax
import jax.numpy as jnp
from jax.experimental import pallas as pl
from jax.experimental.pallas import tpu as pltpu

N = 10000
G = 512
D = 80
NPAD = 10240
CHUNK = 512
DAUG = 88


def _pool_mlp_body(batch_ref, haug_ref, W1_ref, b1_ref, W2_ref, b2_ref,
                   W3_ref, b3_ref, out_ref):
    def step(c, acc):
        bc = batch_ref[pl.ds(c * CHUNK, CHUNK)]
        g_iota = jax.lax.broadcasted_iota(jnp.int32, (G, CHUNK), 0)
        onehot = (bc[None, :] == g_iota).astype(jnp.float32)
        hc = haug_ref[pl.ds(c * CHUNK, CHUNK), :]
        return acc + jnp.dot(onehot, hc, preferred_element_type=jnp.float32)

    pooled_aug = jax.lax.fori_loop(
        0, NPAD // CHUNK, step, jnp.zeros((G, DAUG), jnp.float32))
    cnt = pooled_aug[:, D:D + 1]
    pooled = pooled_aug[:, :D] / jnp.maximum(cnt, 1.0)
    z = jax.nn.relu(jnp.dot(pooled, W1_ref[...],
                            preferred_element_type=jnp.float32) + b1_ref[...])
    z = jax.nn.relu(jnp.dot(z, W2_ref[...],
                            preferred_element_type=jnp.float32) + b2_ref[...])
    out_ref[...] = (jnp.dot(z, W3_ref[...],
                            preferred_element_type=jnp.float32) + b3_ref[...])


@jax.jit
def _pool_mlp(h, batch, W1, b1, W2, b2, W3, b3):
    haug = jnp.concatenate(
        [h, jnp.ones((N, 1), jnp.float32), jnp.zeros((N, DAUG - D - 1), jnp.float32)],
        axis=1)
    haug = jnp.pad(haug, ((0, NPAD - N), (0, 0)))
    batch_pad = jnp.pad(batch, (0, NPAD - N), constant_values=G + 7)
    return pl.pallas_call(
        _pool_mlp_body,
        out_shape=jax.ShapeDtypeStruct((G, 1), jnp.float32),
    )(batch_pad, haug, W1, b1, W2, b2, W3, b3)


def kernel(x, edge_index, edge_attr, batch, atom_tables, We, be, Wp, bp,
           W1, b1, W2, b2, W3, b3):
    h = jnp.zeros((x.shape[0], atom_tables.shape[-1]), dtype=jnp.float32)
    for i in range(9):
        h = h + atom_tables[i][x[:, i]]
    src = edge_index[0]
    dst = edge_index[1]
    n = h.shape[0]
    deg = jax.ops.segment_sum(jnp.ones((src.shape[0],), dtype=jnp.float32), dst, n)
    avg_log = jnp.mean(jnp.log(deg + 1.0))
    L = We.shape[0]
    denom = jnp.maximum(deg, 1.0)[:, None]
    has = (deg > 0)[:, None]
    dlog = jnp.log(deg + 1.0)
    amp = (dlog / avg_log)[:, None]
    att = (avg_log / jnp.maximum(dlog, 1e-5))[:, None]
    for l in range(L):
        e_emb = edge_attr @ We[l] + be[l]
        m = jax.nn.relu(h[dst] + h[src] + e_emb)
        mean = jax.ops.segment_sum(m, dst, n) / denom
        mean_sq = jax.ops.segment_sum(m * m, dst, n) / denom
        std = jnp.sqrt(jax.nn.relu(mean_sq - mean * mean) + 1e-5)
        mn = jnp.where(has, jax.ops.segment_min(m, dst, n), 0.0)
        mx = jnp.where(has, jax.ops.segment_max(m, dst, n), 0.0)
        agg = jnp.concatenate([mean, mn, mx, std], axis=-1)
        out = jnp.concatenate([agg, agg * amp, agg * att], axis=-1)
        k = out @ Wp[l] + bp[l]
        h = jax.nn.relu(k) + h
    return _pool_mlp(h, batch, W1, b1, W2, b2, W3, b3)


# speedup vs baseline: 2.5709x; 2.5501x over previous
import jax
import jax.numpy as jnp
from jax import lax
from jax.experimental import pallas as pl
from jax.experimental.pallas import tpu as pltpu
from jax.experimental.pallas import tpu_sc as plsc

N = 10000
E = 640000
G = 512
D = 80
NPAD = 10240
CHUNK = 512
DAUG = 88

NW = 32
NPW = NPAD // NW
EC_ = 32
EPAD = E + 1088
ACCR = NPW + 8
_SC_MESH = plsc.VectorSubcoreMesh(core_axis_name="c", subcore_axis_name="s")
_SC_PARAMS = pltpu.CompilerParams(use_tc_tiling_on_sc=False)
_HI = jax.lax.Precision.HIGHEST


_ENC_NC = 16


def _encoder_body(idx_hbm, table_hbm, h_hbm, idx_v, rows_v, acc_v, sem):
    w = lax.axis_index("c") * 16 + lax.axis_index("s")

    def chunk(ci, carry):
        pltpu.sync_copy(
            idx_hbm.at[pl.ds(w * (NPW * 9) + ci * (_ENC_NC * 9), _ENC_NC * 9)],
            idx_v)
        pltpu.async_copy(table_hbm.at[idx_v], rows_v, sem).wait()
        for n in range(_ENC_NC):
            for v in range(D // 16):
                sl = pl.ds(v * 16, 16)
                a = rows_v[n * 9, sl]
                for i in range(1, 9):
                    a = a + rows_v[n * 9 + i, sl]
                acc_v[n, sl] = a
        pltpu.sync_copy(acc_v,
                        h_hbm.at[pl.ds(w * NPW + ci * _ENC_NC, _ENC_NC), :])
        return carry

    lax.fori_loop(0, NPW // _ENC_NC, chunk, 0)


def _encoder(x_pad, atom_tables):
    idx = (x_pad + (jnp.arange(9, dtype=jnp.int32) * 128)[None, :]).reshape(-1)
    table = atom_tables.reshape(9 * 128, D)
    kfn = pl.kernel(
        _encoder_body,
        out_type=jax.ShapeDtypeStruct((NPAD, D), jnp.float32),
        mesh=_SC_MESH,
        compiler_params=_SC_PARAMS,
        scratch_types=[
            pltpu.VMEM((_ENC_NC * 9,), jnp.int32),
            pltpu.VMEM((_ENC_NC * 9, D), jnp.float32),
            pltpu.VMEM((_ENC_NC, D), jnp.float32),
            pltpu.SemaphoreType.DMA,
        ],
    )
    return kfn(idx, table)


def _make_edge_body(with_deg):
    def body(pk_hbm, scal_hbm, h_hbm, ee_hbm,
             sum_o, sq_o, mn_o, mx_o, *rest):
        if with_deg:
            deg_o = rest[0]
            rest = rest[1:]
        (asum, asq, amn, amx, acnt, hs_v, hd_v, ee_v, pkv, srcv, eidv,
         dlmv, dstgv, scal_v, sg0, sg1, si0, si1) = rest
        sg = (sg0, sg1)
        si = (si0, si1)
        c = lax.axis_index("c")
        s = lax.axis_index("s")
        b = c * 16 + s
        pltpu.sync_copy(scal_hbm, scal_v)
        rowv = scal_v[b, :]
        bo_b = pl.multiple_of(rowv[0], EC_)
        ec_b = pl.multiple_of(rowv[1], EC_)
        limit = bo_b + ec_b
        nch = (ec_b + (EC_ - 1)) // EC_
        nsuper = (nch + 1) // 2

        zf = jnp.zeros((16,), jnp.float32)
        big = jnp.full((16,), 1e30, jnp.float32)
        ones16 = jnp.ones((16,), jnp.float32)
        iota = lax.iota(jnp.int32, 16)

        def initloop(i, carry):
            o = pl.ds(i * 16, 16)
            asum[o] = zf
            asq[o] = zf
            amn[o] = big
            amx[o] = zf
            return carry

        lax.fori_loop(0, (ACCR * D) // 16, initloop, 0)
        if with_deg:
            def initcnt(i, carry):
                acnt[pl.ds(i * 16, 16)] = zf
                return carry
            lax.fori_loop(0, ACCR, initcnt, 0)

        def fire_idx(j, p):
            off = (bo_b + j * EC_) * 3
            pltpu.async_copy(pk_hbm.at[pl.ds(off, 3 * EC_)], pkv.at[p], si[p])

        def wait_idx(p):
            pltpu.make_async_copy(pk_hbm.at[pl.ds(0, 3 * EC_)], pkv.at[p],
                                  si[p]).wait()

        def build_and_fire(j, p):
            base = bo_b + j * EC_
            for g in range(EC_ // 16):
                sl = pl.ds(g * 16, 16)
                gi = iota + (base + g * 16)
                msk = gi < limit
                dl_raw = pkv[p, pl.ds(EC_ + g * 16, 16)]
                dlmv[p, sl] = jnp.where(msk, dl_raw, NPW)
                dstgv[p, sl] = jnp.minimum(b * NPW + dl_raw, NPAD - 1)
                srcv[p, sl] = pkv[p, pl.ds(g * 16, 16)]
                eidv[p, sl] = pkv[p, pl.ds(2 * EC_ + g * 16, 16)]
            pltpu.async_copy(h_hbm.at[srcv.at[p]], hs_v.at[p], sg[p])
            pltpu.async_copy(h_hbm.at[dstgv.at[p]], hd_v.at[p], sg[p])
            pltpu.async_copy(ee_hbm.at[eidv.at[p]], ee_v.at[p], sg[p])

        def wait_gathers(p):
            pltpu.make_async_copy(h_hbm.at[srcv.at[p]], hs_v.at[p],
                                  sg[p]).wait()
            pltpu.make_async_copy(h_hbm.at[dstgv.at[p]], hd_v.at[p],
                                  sg[p]).wait()
            pltpu.make_async_copy(ee_hbm.at[eidv.at[p]], ee_v.at[p],
                                  sg[p]).wait()

        def compute(p):
            for g in range(EC_ // 16):
                dlm16 = dlmv[p, pl.ds(g * 16, 16)]
                for k in range(16):
                    e = g * 16 + k
                    off = dlm16[k] * D
                    for v in range(D // 16):
                        sl = pl.ds(v * 16, 16)
                        o = pl.ds(off + v * 16, 16)
                        m = jnp.maximum(
                            hs_v[p, e, sl] + hd_v[p, e, sl] + ee_v[p, e, sl],
                            0.0)
                        plsc.addupdate(asum.at[o], m)
                        plsc.addupdate(asq.at[o], m * m)
                        amn[o] = jnp.minimum(amn[o], m)
                        amx[o] = jnp.maximum(amx[o], m)
                    if with_deg:
                        plsc.addupdate(acnt.at[pl.ds(dlm16[k] * 16, 16)],
                                       ones16)

        @pl.when(nch > 0)
        def _():
            fire_idx(0, 0)

        @pl.when(nch > 1)
        def _():
            fire_idx(1, 1)

        @pl.when(nch > 0)
        def _():
            wait_idx(0)
            build_and_fire(0, 0)

        def super_body(jj, carry):
            for p in (0, 1):
                j = 2 * jj + p
                q = 1 - p

                @pl.when(j + 1 < nch)
                def _():
                    wait_idx(q)
                    build_and_fire(j + 1, q)

                @pl.when(j + 2 < nch)
                def _():
                    fire_idx(j + 2, p)

                @pl.when(j < nch)
                def _():
                    wait_gathers(p)
                    compute(p)
            return carry

        lax.fori_loop(0, nsuper, super_body, 0)

        nb = b * (NPW * D)
        pltpu.sync_copy(asum.at[pl.ds(0, NPW * D)],
                        sum_o.at[pl.ds(nb, NPW * D)])
        pltpu.sync_copy(asq.at[pl.ds(0, NPW * D)],
                        sq_o.at[pl.ds(nb, NPW * D)])
        pltpu.sync_copy(amn.at[pl.ds(0, NPW * D)],
                        mn_o.at[pl.ds(nb, NPW * D)])
        pltpu.sync_copy(amx.at[pl.ds(0, NPW * D)],
                        mx_o.at[pl.ds(nb, NPW * D)])
        if with_deg:
            pltpu.sync_copy(acnt.at[pl.ds(0, NPW * 16)],
                            deg_o.at[pl.ds(b * NPW * 16, NPW * 16)])

    body.__name__ = f"edge_body_deg{int(with_deg)}"
    return body


def _edge_kernel(with_deg):
    outs = [jax.ShapeDtypeStruct((NPAD * D,), jnp.float32)] * 4
    if with_deg:
        outs = outs + [jax.ShapeDtypeStruct((NPAD * 16,), jnp.float32)]
    return pl.kernel(
        _make_edge_body(with_deg),
        out_type=outs,
        mesh=_SC_MESH,
        compiler_params=_SC_PARAMS,
        scratch_types=[
            pltpu.VMEM((ACCR * D,), jnp.float32),
            pltpu.VMEM((ACCR * D,), jnp.float32),
            pltpu.VMEM((ACCR * D,), jnp.float32),
            pltpu.VMEM((ACCR * D,), jnp.float32),
            pltpu.VMEM((ACCR * 16,), jnp.float32),
            pltpu.VMEM((2, EC_, D), jnp.float32),
            pltpu.VMEM((2, EC_, D), jnp.float32),
            pltpu.VMEM((2, EC_, D), jnp.float32),
            pltpu.VMEM((2, 3 * EC_), jnp.int32),
            pltpu.VMEM((2, EC_), jnp.int32),
            pltpu.VMEM((2, EC_), jnp.int32),
            pltpu.VMEM((2, EC_), jnp.int32),
            pltpu.VMEM((2, EC_), jnp.int32),
            pltpu.VMEM((NW, 16), jnp.int32),
            pltpu.SemaphoreType.DMA,
            pltpu.SemaphoreType.DMA,
            pltpu.SemaphoreType.DMA,
            pltpu.SemaphoreType.DMA,
        ],
    )


def _eemb_body(ea_ref, we_ref, be_ref, o0, o1, o2, o3):
    z = jnp.dot(ea_ref[...], we_ref[...],
                preferred_element_type=jnp.float32) + be_ref[...]
    o0[...] = z[:, 0 * D:1 * D]
    o1[...] = z[:, 1 * D:2 * D]
    o2[...] = z[:, 2 * D:3 * D]
    o3[...] = z[:, 3 * D:4 * D]


def _eemb(edge_attr, We, be):
    wec = We.transpose(1, 0, 2).reshape(16, 4 * D)
    bec = be.reshape(1, 4 * D)
    BL = 2560
    outs = [jax.ShapeDtypeStruct((E, D), jnp.float32)] * 4
    return pl.pallas_call(
        _eemb_body,
        grid=(E // BL,),
        in_specs=[
            pl.BlockSpec((BL, 16), lambda i: (i, 0)),
            pl.BlockSpec((16, 4 * D), lambda i: (0, 0)),
            pl.BlockSpec((1, 4 * D), lambda i: (0, 0)),
        ],
        out_specs=[pl.BlockSpec((BL, D), lambda i: (i, 0))] * 4,
        out_shape=outs,
    )(edge_attr, wec, bec)


def _avgl_body(deg_ref, out_ref):
    d = deg_ref[:, 0:1]
    r = lax.broadcasted_iota(jnp.int32, d.shape, 0)
    valid = r < N
    out_ref[...] = (jnp.sum(jnp.where(valid, jnp.log(d + 1.0), 0.0))
                    / N).reshape(1, 1)


def _avgl(deg16):
    return pl.pallas_call(
        _avgl_body,
        grid=(1,),
        in_specs=[pl.BlockSpec((NPAD, 16), lambda i: (0, 0))],
        out_specs=pl.BlockSpec((1, 1), lambda i: (0, 0)),
        out_shape=jax.ShapeDtypeStruct((1, 1), jnp.float32),
    )(deg16)


def _node_body(avgl_ref, deg_ref, sum_ref, sq_ref, mn_ref, mx_ref, h_ref,
               wp_ref, bp_ref, out_ref):
    avgl = avgl_ref[0, 0]
    deg = deg_ref[:, 0:1]
    dlog = jnp.log(deg + 1.0)
    amp = dlog / avgl
    att = avgl / jnp.maximum(dlog, 1e-5)
    denom = jnp.maximum(deg, 1.0)
    has = deg > 0
    mean = sum_ref[...] / denom
    msq = sq_ref[...] / denom
    std = jnp.sqrt(jax.nn.relu(msq - mean * mean) + 1e-5)
    mn = jnp.where(has, mn_ref[...], 0.0)
    mx = jnp.where(has, mx_ref[...], 0.0)
    agg = jnp.concatenate([mean, mn, mx, std], axis=-1)
    full = jnp.concatenate([agg, agg * amp, agg * att], axis=-1)
    z = jnp.dot(full, wp_ref[...],
                preferred_element_type=jnp.float32) + bp_ref[...]
    out_ref[...] = jax.nn.relu(z) + h_ref[...]


def _node(avgl, deg, ssum, ssq, smn, smx, h, Wp_l, bp_l):
    BL = 256
    return pl.pallas_call(
        _node_body,
        grid=(NPAD // BL,),
        in_specs=[
            pl.BlockSpec((1, 1), lambda i: (0, 0)),
            pl.BlockSpec((BL, 16), lambda i: (i, 0)),
            pl.BlockSpec((BL, D), lambda i: (i, 0)),
            pl.BlockSpec((BL, D), lambda i: (i, 0)),
            pl.BlockSpec((BL, D), lambda i: (i, 0)),
            pl.BlockSpec((BL, D), lambda i: (i, 0)),
            pl.BlockSpec((BL, D), lambda i: (i, 0)),
            pl.BlockSpec((12 * D, D), lambda i: (0, 0)),
            pl.BlockSpec((1, D), lambda i: (0, 0)),
        ],
        out_specs=pl.BlockSpec((BL, D), lambda i: (i, 0)),
        out_shape=jax.ShapeDtypeStruct((NPAD, D), jnp.float32),
    )(avgl, deg, ssum.reshape(NPAD, D), ssq.reshape(NPAD, D),
      smn.reshape(NPAD, D), smx.reshape(NPAD, D), h, Wp_l,
      bp_l.reshape(1, D))


def _pool_mlp_body(batch_ref, haug_ref, W1_ref, b1_ref, W2_ref, b2_ref,
                   W3_ref, b3_ref, out_ref):
    def step(c, acc):
        bc = batch_ref[pl.ds(c * CHUNK, CHUNK)]
        g_iota = lax.broadcasted_iota(jnp.int32, (G, CHUNK), 0)
        onehot = (bc[None, :] == g_iota).astype(jnp.float32)
        hc = haug_ref[pl.ds(c * CHUNK, CHUNK), :]
        return acc + jnp.dot(onehot, hc, preferred_element_type=jnp.float32,
                             precision=_HI)

    pooled_aug = lax.fori_loop(
        0, NPAD // CHUNK, step, jnp.zeros((G, DAUG), jnp.float32))
    cnt = pooled_aug[:, D:D + 1]
    pooled = pooled_aug[:, :D] / jnp.maximum(cnt, 1.0)
    z = jax.nn.relu(jnp.dot(pooled, W1_ref[...],
                            preferred_element_type=jnp.float32) + b1_ref[...])
    z = jax.nn.relu(jnp.dot(z, W2_ref[...],
                            preferred_element_type=jnp.float32) + b2_ref[...])
    out_ref[...] = (jnp.dot(z, W3_ref[...],
                            preferred_element_type=jnp.float32) + b3_ref[...])


def _pool_mlp(h_pad, batch, W1, b1, W2, b2, W3, b3):
    mask = (jnp.arange(NPAD, dtype=jnp.int32) < N)[:, None]
    haug = jnp.concatenate(
        [h_pad, mask.astype(jnp.float32),
         jnp.zeros((NPAD, DAUG - D - 1), jnp.float32)], axis=1)
    haug = jnp.where(mask, haug, 0.0)
    batch_pad = jnp.pad(batch, (0, NPAD - N), constant_values=G + 7)
    return pl.pallas_call(
        _pool_mlp_body,
        out_shape=jax.ShapeDtypeStruct((G, 1), jnp.float32),
    )(batch_pad, haug, W1, b1, W2, b2, W3, b3)


def _preprocess(src, dst):
    b = dst // NPW
    cnt = jnp.bincount(b, length=NW).astype(jnp.int32)
    c32 = (cnt + 31) // 32 * 32
    bo = jnp.concatenate([jnp.zeros((1,), jnp.int32), jnp.cumsum(c32)])
    boe = jnp.concatenate([jnp.zeros((1,), jnp.int32), jnp.cumsum(cnt)])
    r = jnp.argsort(b, stable=True).astype(jnp.int32)
    br = b[r]
    pos = bo[br] + jnp.arange(E, dtype=jnp.int32) - boe[br]
    src_p = jnp.zeros((EPAD,), jnp.int32).at[pos].set(src[r])
    dl_p = jnp.full((EPAD,), NPW, jnp.int32).at[pos].set(dst[r] - br * NPW)
    eid_p = jnp.zeros((EPAD,), jnp.int32).at[pos].set(r)
    pk = jnp.concatenate(
        [src_p.reshape(-1, EC_), dl_p.reshape(-1, EC_),
         eid_p.reshape(-1, EC_)], axis=1).reshape(-1)
    scal = jnp.stack([bo[:32], c32], axis=1)
    scal = jnp.pad(scal, ((0, 0), (0, 14)))
    return pk, scal


def kernel(x, edge_index, edge_attr, batch, atom_tables, We, be, Wp, bp,
           W1, b1, W2, b2, W3, b3):
    x_pad = jnp.pad(x, ((0, NPAD - N), (0, 0)))
    h = _encoder(x_pad, atom_tables)
    src = edge_index[0]
    dst = edge_index[1]
    pk, scal = _preprocess(src, dst)
    ees = _eemb(edge_attr, We, be)

    edge0 = _edge_kernel(True)
    edgeL = _edge_kernel(False)

    ssum, ssq, smn, smx, deg16 = edge0(pk, scal, h, ees[0])
    deg16 = deg16.reshape(NPAD, 16)
    avgl = _avgl(deg16)
    h = _node(avgl, deg16, ssum, ssq, smn, smx, h, Wp[0], bp[0])
    for l in range(1, 4):
        ssum, ssq, smn, smx = edgeL(pk, scal, h, ees[l])
        h = _node(avgl, deg16, ssum, ssq, smn, smx, h, Wp[l], bp[l])
    return _pool_mlp(h, batch, W1, b1, W2, b2, W3, b3)


# speedup vs baseline: 4.2593x; 1.6567x over previous
import jax
import jax.numpy as jnp
from jax import lax
from jax.experimental import pallas as pl
from jax.experimental.pallas import tpu as pltpu
from jax.experimental.pallas import tpu_sc as plsc

N = 10000
E = 640000
G = 512
D = 80
NPAD = 10240
CHUNK = 512
DAUG = 88

NW = 32
NPW = NPAD // NW
EC_ = 32
EPAD = E + 64
ACCR = NPW + 8
_SC_MESH = plsc.VectorSubcoreMesh(core_axis_name="c", subcore_axis_name="s")
_SC_PARAMS = pltpu.CompilerParams(use_tc_tiling_on_sc=False)
_HI = jax.lax.Precision.HIGHEST


_ENC_NC = 16


def _encoder_body(idx_hbm, table_hbm, h_hbm, idx_v, rows_v, acc_v, sem):
    w = lax.axis_index("c") * 16 + lax.axis_index("s")

    def chunk(ci, carry):
        pltpu.sync_copy(
            idx_hbm.at[pl.ds(w * (NPW * 9) + ci * (_ENC_NC * 9), _ENC_NC * 9)],
            idx_v)
        pltpu.async_copy(table_hbm.at[idx_v], rows_v, sem).wait()
        for n in range(_ENC_NC):
            for v in range(D // 16):
                sl = pl.ds(v * 16, 16)
                a = rows_v[n * 9, sl]
                for i in range(1, 9):
                    a = a + rows_v[n * 9 + i, sl]
                acc_v[n, sl] = a
        pltpu.sync_copy(acc_v,
                        h_hbm.at[pl.ds(w * NPW + ci * _ENC_NC, _ENC_NC), :])
        return carry

    lax.fori_loop(0, NPW // _ENC_NC, chunk, 0)


def _encoder(x_pad, atom_tables):
    idx = (x_pad + (jnp.arange(9, dtype=jnp.int32) * 128)[None, :]).reshape(-1)
    table = atom_tables.reshape(9 * 128, D)
    kfn = pl.kernel(
        _encoder_body,
        out_type=jax.ShapeDtypeStruct((NPAD, D), jnp.float32),
        mesh=_SC_MESH,
        compiler_params=_SC_PARAMS,
        scratch_types=[
            pltpu.VMEM((_ENC_NC * 9,), jnp.int32),
            pltpu.VMEM((_ENC_NC * 9, D), jnp.float32),
            pltpu.VMEM((_ENC_NC, D), jnp.float32),
            pltpu.SemaphoreType.DMA,
        ],
    )
    return kfn(idx, table)


def _make_edge_body(with_deg):
    def body(pk_hbm, scal_hbm, h_hbm, ee_hbm,
             sum_o, sq_o, mn_o, mx_o, *rest):
        if with_deg:
            deg_o = rest[0]
            rest = rest[1:]
        (asum, asq, amn, amx, acnt, hs_v, hd_v, ee_v, pkv, srcv, eidv,
         dlmv, dstgv, scal_v, sg0, sg1, si0, si1) = rest
        sg = (sg0, sg1)
        si = (si0, si1)
        c = lax.axis_index("c")
        s = lax.axis_index("s")
        b = c * 16 + s
        pltpu.sync_copy(scal_hbm, scal_v)
        rowv = scal_v[b, :]
        bo_b = rowv[0]
        ec_b = rowv[1]
        limit = bo_b + ec_b
        start = pl.multiple_of((bo_b // EC_) * EC_, EC_)
        nch = (limit - start + (EC_ - 1)) // EC_
        nsuper = (nch + 1) // 2

        zf = jnp.zeros((16,), jnp.float32)
        big = jnp.full((16,), 1e30, jnp.float32)
        ones16 = jnp.ones((16,), jnp.float32)
        iota = lax.iota(jnp.int32, 16)

        def initloop(i, carry):
            o = pl.ds(i * 16, 16)
            asum[o] = zf
            asq[o] = zf
            amn[o] = big
            amx[o] = zf
            return carry

        lax.fori_loop(0, (ACCR * D) // 16, initloop, 0)
        if with_deg:
            def initcnt(i, carry):
                acnt[pl.ds(i * 16, 16)] = zf
                return carry
            lax.fori_loop(0, ACCR, initcnt, 0)

        def fire_idx(j, p):
            off = (start + j * EC_) * 3
            pltpu.async_copy(pk_hbm.at[pl.ds(off, 3 * EC_)], pkv.at[p], si[p])

        def wait_idx(p):
            pltpu.make_async_copy(pk_hbm.at[pl.ds(0, 3 * EC_)], pkv.at[p],
                                  si[p]).wait()

        def build_and_fire(j, p):
            base = start + j * EC_
            for g in range(EC_ // 16):
                sl = pl.ds(g * 16, 16)
                gi = iota + (base + g * 16)
                msk = (gi >= bo_b) & (gi < limit)
                dl_raw = pkv[p, pl.ds(EC_ + g * 16, 16)]
                dlmv[p, sl] = jnp.where(msk, dl_raw, NPW)
                dstgv[p, sl] = jnp.minimum(b * NPW + dl_raw, NPAD - 1)
                srcv[p, sl] = pkv[p, pl.ds(g * 16, 16)]
                eidv[p, sl] = pkv[p, pl.ds(2 * EC_ + g * 16, 16)]
            pltpu.async_copy(h_hbm.at[srcv.at[p]], hs_v.at[p], sg[p])
            pltpu.async_copy(h_hbm.at[dstgv.at[p]], hd_v.at[p], sg[p])
            pltpu.async_copy(ee_hbm.at[eidv.at[p]], ee_v.at[p], sg[p])

        def wait_gathers(p):
            pltpu.make_async_copy(h_hbm.at[srcv.at[p]], hs_v.at[p],
                                  sg[p]).wait()
            pltpu.make_async_copy(h_hbm.at[dstgv.at[p]], hd_v.at[p],
                                  sg[p]).wait()
            pltpu.make_async_copy(ee_hbm.at[eidv.at[p]], ee_v.at[p],
                                  sg[p]).wait()

        def compute(p):
            for g in range(EC_ // 16):
                dlm16 = dlmv[p, pl.ds(g * 16, 16)]
                for k in range(16):
                    e = g * 16 + k
                    off = dlm16[k] * D
                    for v in range(D // 16):
                        sl = pl.ds(v * 16, 16)
                        o = pl.ds(off + v * 16, 16)
                        m = jnp.maximum(
                            hs_v[p, e, sl] + hd_v[p, e, sl] + ee_v[p, e, sl],
                            0.0)
                        plsc.addupdate(asum.at[o], m)
                        plsc.addupdate(asq.at[o], m * m)
                        amn[o] = jnp.minimum(amn[o], m)
                        amx[o] = jnp.maximum(amx[o], m)
                    if with_deg:
                        plsc.addupdate(acnt.at[pl.ds(dlm16[k] * 16, 16)],
                                       ones16)

        @pl.when(nch > 0)
        def _():
            fire_idx(0, 0)

        @pl.when(nch > 1)
        def _():
            fire_idx(1, 1)

        @pl.when(nch > 0)
        def _():
            wait_idx(0)
            build_and_fire(0, 0)

        def super_body(jj, carry):
            for p in (0, 1):
                j = 2 * jj + p
                q = 1 - p

                @pl.when(j + 1 < nch)
                def _():
                    wait_idx(q)
                    build_and_fire(j + 1, q)

                @pl.when(j + 2 < nch)
                def _():
                    fire_idx(j + 2, p)

                @pl.when(j < nch)
                def _():
                    wait_gathers(p)
                    compute(p)
            return carry

        lax.fori_loop(0, nsuper, super_body, 0)

        nb = b * (NPW * D)
        pltpu.sync_copy(asum.at[pl.ds(0, NPW * D)],
                        sum_o.at[pl.ds(nb, NPW * D)])
        pltpu.sync_copy(asq.at[pl.ds(0, NPW * D)],
                        sq_o.at[pl.ds(nb, NPW * D)])
        pltpu.sync_copy(amn.at[pl.ds(0, NPW * D)],
                        mn_o.at[pl.ds(nb, NPW * D)])
        pltpu.sync_copy(amx.at[pl.ds(0, NPW * D)],
                        mx_o.at[pl.ds(nb, NPW * D)])
        if with_deg:
            pltpu.sync_copy(acnt.at[pl.ds(0, NPW * 16)],
                            deg_o.at[pl.ds(b * NPW * 16, NPW * 16)])

    body.__name__ = f"edge_body_deg{int(with_deg)}"
    return body


def _edge_kernel(with_deg):
    outs = [jax.ShapeDtypeStruct((NPAD * D,), jnp.float32)] * 4
    if with_deg:
        outs = outs + [jax.ShapeDtypeStruct((NPAD * 16,), jnp.float32)]
    return pl.kernel(
        _make_edge_body(with_deg),
        out_type=outs,
        mesh=_SC_MESH,
        compiler_params=_SC_PARAMS,
        scratch_types=[
            pltpu.VMEM((ACCR * D,), jnp.float32),
            pltpu.VMEM((ACCR * D,), jnp.float32),
            pltpu.VMEM((ACCR * D,), jnp.float32),
            pltpu.VMEM((ACCR * D,), jnp.float32),
            pltpu.VMEM((ACCR * 16,), jnp.float32),
            pltpu.VMEM((2, EC_, D), jnp.float32),
            pltpu.VMEM((2, EC_, D), jnp.float32),
            pltpu.VMEM((2, EC_, D), jnp.float32),
            pltpu.VMEM((2, 3 * EC_), jnp.int32),
            pltpu.VMEM((2, EC_), jnp.int32),
            pltpu.VMEM((2, EC_), jnp.int32),
            pltpu.VMEM((2, EC_), jnp.int32),
            pltpu.VMEM((2, EC_), jnp.int32),
            pltpu.VMEM((NW, 16), jnp.int32),
            pltpu.SemaphoreType.DMA,
            pltpu.SemaphoreType.DMA,
            pltpu.SemaphoreType.DMA,
            pltpu.SemaphoreType.DMA,
        ],
    )


def _eemb_body(ea_ref, we_ref, be_ref, o0, o1, o2, o3):
    z = jnp.dot(ea_ref[...], we_ref[...],
                preferred_element_type=jnp.float32) + be_ref[...]
    o0[...] = z[:, 0 * D:1 * D]
    o1[...] = z[:, 1 * D:2 * D]
    o2[...] = z[:, 2 * D:3 * D]
    o3[...] = z[:, 3 * D:4 * D]


def _eemb(edge_attr, We, be):
    wec = We.transpose(1, 0, 2).reshape(16, 4 * D)
    bec = be.reshape(1, 4 * D)
    BL = 2560
    outs = [jax.ShapeDtypeStruct((E, D), jnp.float32)] * 4
    return pl.pallas_call(
        _eemb_body,
        grid=(E // BL,),
        in_specs=[
            pl.BlockSpec((BL, 16), lambda i: (i, 0)),
            pl.BlockSpec((16, 4 * D), lambda i: (0, 0)),
            pl.BlockSpec((1, 4 * D), lambda i: (0, 0)),
        ],
        out_specs=[pl.BlockSpec((BL, D), lambda i: (i, 0))] * 4,
        out_shape=outs,
    )(edge_attr, wec, bec)


def _avgl_body(deg_ref, out_ref):
    d = deg_ref[:, 0:1]
    r = lax.broadcasted_iota(jnp.int32, d.shape, 0)
    valid = r < N
    out_ref[...] = (jnp.sum(jnp.where(valid, jnp.log(d + 1.0), 0.0))
                    / N).reshape(1, 1)


def _avgl(deg16):
    return pl.pallas_call(
        _avgl_body,
        grid=(1,),
        in_specs=[pl.BlockSpec((NPAD, 16), lambda i: (0, 0))],
        out_specs=pl.BlockSpec((1, 1), lambda i: (0, 0)),
        out_shape=jax.ShapeDtypeStruct((1, 1), jnp.float32),
    )(deg16)


def _node_body(avgl_ref, deg_ref, sum_ref, sq_ref, mn_ref, mx_ref, h_ref,
               wp_ref, bp_ref, out_ref):
    avgl = avgl_ref[0, 0]
    deg = deg_ref[:, 0:1]
    dlog = jnp.log(deg + 1.0)
    amp = dlog / avgl
    att = avgl / jnp.maximum(dlog, 1e-5)
    denom = jnp.maximum(deg, 1.0)
    has = deg > 0
    mean = sum_ref[...] / denom
    msq = sq_ref[...] / denom
    std = jnp.sqrt(jax.nn.relu(msq - mean * mean) + 1e-5)
    mn = jnp.where(has, mn_ref[...], 0.0)
    mx = jnp.where(has, mx_ref[...], 0.0)
    agg = jnp.concatenate([mean, mn, mx, std], axis=-1)
    full = jnp.concatenate([agg, agg * amp, agg * att], axis=-1)
    z = jnp.dot(full, wp_ref[...],
                preferred_element_type=jnp.float32) + bp_ref[...]
    out_ref[...] = jax.nn.relu(z) + h_ref[...]


def _node(avgl, deg, ssum, ssq, smn, smx, h, Wp_l, bp_l):
    BL = 256
    return pl.pallas_call(
        _node_body,
        grid=(NPAD // BL,),
        in_specs=[
            pl.BlockSpec((1, 1), lambda i: (0, 0)),
            pl.BlockSpec((BL, 16), lambda i: (i, 0)),
            pl.BlockSpec((BL, D), lambda i: (i, 0)),
            pl.BlockSpec((BL, D), lambda i: (i, 0)),
            pl.BlockSpec((BL, D), lambda i: (i, 0)),
            pl.BlockSpec((BL, D), lambda i: (i, 0)),
            pl.BlockSpec((BL, D), lambda i: (i, 0)),
            pl.BlockSpec((12 * D, D), lambda i: (0, 0)),
            pl.BlockSpec((1, D), lambda i: (0, 0)),
        ],
        out_specs=pl.BlockSpec((BL, D), lambda i: (i, 0)),
        out_shape=jax.ShapeDtypeStruct((NPAD, D), jnp.float32),
    )(avgl, deg, ssum.reshape(NPAD, D), ssq.reshape(NPAD, D),
      smn.reshape(NPAD, D), smx.reshape(NPAD, D), h, Wp_l,
      bp_l.reshape(1, D))


def _pool_mlp_body(batch_ref, haug_ref, W1_ref, b1_ref, W2_ref, b2_ref,
                   W3_ref, b3_ref, out_ref):
    def step(c, acc):
        bc = batch_ref[pl.ds(c * CHUNK, CHUNK)]
        g_iota = lax.broadcasted_iota(jnp.int32, (G, CHUNK), 0)
        onehot = (bc[None, :] == g_iota).astype(jnp.float32)
        hc = haug_ref[pl.ds(c * CHUNK, CHUNK), :]
        return acc + jnp.dot(onehot, hc, preferred_element_type=jnp.float32,
                             precision=_HI)

    pooled_aug = lax.fori_loop(
        0, NPAD // CHUNK, step, jnp.zeros((G, DAUG), jnp.float32))
    cnt = pooled_aug[:, D:D + 1]
    pooled = pooled_aug[:, :D] / jnp.maximum(cnt, 1.0)
    z = jax.nn.relu(jnp.dot(pooled, W1_ref[...],
                            preferred_element_type=jnp.float32) + b1_ref[...])
    z = jax.nn.relu(jnp.dot(z, W2_ref[...],
                            preferred_element_type=jnp.float32) + b2_ref[...])
    out_ref[...] = (jnp.dot(z, W3_ref[...],
                            preferred_element_type=jnp.float32) + b3_ref[...])


def _pool_mlp(h_pad, batch, W1, b1, W2, b2, W3, b3):
    mask = (jnp.arange(NPAD, dtype=jnp.int32) < N)[:, None]
    haug = jnp.concatenate(
        [h_pad, mask.astype(jnp.float32),
         jnp.zeros((NPAD, DAUG - D - 1), jnp.float32)], axis=1)
    haug = jnp.where(mask, haug, 0.0)
    batch_pad = jnp.pad(batch, (0, NPAD - N), constant_values=G + 7)
    return pl.pallas_call(
        _pool_mlp_body,
        out_shape=jax.ShapeDtypeStruct((G, 1), jnp.float32),
    )(batch_pad, haug, W1, b1, W2, b2, W3, b3)


def _preprocess(src, dst):
    b = dst // NPW
    eid = jnp.arange(E, dtype=jnp.int32)
    bs, srcs, dsts, eids = lax.sort((b, src, dst, eid), num_keys=1,
                                    is_stable=True)
    dls = dsts - bs * NPW
    cnt = jnp.sum(
        (b[:, None] == jnp.arange(NW, dtype=jnp.int32)[None, :]),
        axis=0, dtype=jnp.int32)
    bo = jnp.concatenate([jnp.zeros((1,), jnp.int32), jnp.cumsum(cnt)])
    src_p = jnp.pad(srcs, (0, EPAD - E))
    dl_p = jnp.pad(dls, (0, EPAD - E), constant_values=NPW)
    eid_p = jnp.pad(eids, (0, EPAD - E))
    pk = jnp.concatenate(
        [src_p.reshape(-1, EC_), dl_p.reshape(-1, EC_),
         eid_p.reshape(-1, EC_)], axis=1).reshape(-1)
    scal = jnp.stack([bo[:32], cnt], axis=1)
    scal = jnp.pad(scal, ((0, 0), (0, 14)))
    return pk, scal


def kernel(x, edge_index, edge_attr, batch, atom_tables, We, be, Wp, bp,
           W1, b1, W2, b2, W3, b3):
    x_pad = jnp.pad(x, ((0, NPAD - N), (0, 0)))
    h = _encoder(x_pad, atom_tables)
    src = edge_index[0]
    dst = edge_index[1]
    pk, scal = _preprocess(src, dst)
    ees = _eemb(edge_attr, We, be)

    edge0 = _edge_kernel(True)
    edgeL = _edge_kernel(False)

    ssum, ssq, smn, smx, deg16 = edge0(pk, scal, h, ees[0])
    deg16 = deg16.reshape(NPAD, 16)
    avgl = _avgl(deg16)
    h = _node(avgl, deg16, ssum, ssq, smn, smx, h, Wp[0], bp[0])
    for l in range(1, 4):
        ssum, ssq, smn, smx = edgeL(pk, scal, h, ees[l])
        h = _node(avgl, deg16, ssum, ssq, smn, smx, h, Wp[l], bp[l])
    return _pool_mlp(h, batch, W1, b1, W2, b2, W3, b3)
